# baseline (speedup 1.0000x reference)
import dataclasses

import jax
import jax.numpy as jnp
from jax import lax
from jax.experimental import pallas as pl
from jax.experimental.pallas import tpu as pltpu
from jax.experimental.pallas import tpu_sc as plsc

N_ROWS = 65536
N_COLS = 65536
NNZ_PER_ROW = 64

NUM_WORKERS = 32
ROWS_PER_WORKER = N_ROWS // NUM_WORKERS
CHUNK_ROWS = 128
NUM_CHUNKS = ROWS_PER_WORKER // CHUNK_ROWS
CHUNK_NNZ = CHUNK_ROWS * NNZ_PER_ROW
LANES = 16
IDX_PER_STREAM = 128
GSTREAMS = CHUNK_NNZ // IDX_PER_STREAM


def _spmv_kernel(table_hbm, cols_hbm, values_hbm, bias_hbm, out_hbm,
                 table_sh, cols_v, values_v, gath_v, out_v,
                 sem_t, csem0, csem1, vsem0, vsem1, gsem0, gsem1):
    csem = (csem0, csem1)
    vsem = (vsem0, vsem1)
    gsem = (gsem0, gsem1)
    wid = lax.axis_index("s") * 2 + lax.axis_index("c")
    base_row = wid * ROWS_PER_WORKER

    def hbm_slice(ref, c):
        nz0 = (base_row + c * CHUNK_ROWS) * NNZ_PER_ROW
        return ref.at[pl.ds(nz0, CHUNK_NNZ)]

    def buf_slice(ref, b):
        return ref.at[pl.ds(b * CHUNK_NNZ, CHUNK_NNZ)]

    def start_cols(c, b):
        pltpu.async_copy(hbm_slice(cols_hbm, c), buf_slice(cols_v, b),
                         csem[b])

    def wait_cols(c, b):
        pltpu.make_async_copy(hbm_slice(cols_hbm, c), buf_slice(cols_v, b),
                              csem[b]).wait()

    def start_values(c, b):
        pltpu.async_copy(hbm_slice(values_hbm, c), buf_slice(values_v, b),
                         vsem[b])

    def wait_values(c, b):
        pltpu.make_async_copy(hbm_slice(values_hbm, c), buf_slice(values_v, b),
                              vsem[b]).wait()

    def gather_copy(b, j):
        off = b * CHUNK_NNZ + j * IDX_PER_STREAM
        idx = cols_v.at[pl.ds(off, IDX_PER_STREAM)]
        return pltpu.make_async_copy(table_sh.at[idx],
                                     gath_v.at[pl.ds(off, IDX_PER_STREAM)],
                                     gsem[b])

    def issue_gathers(b):
        @pl.loop(0, GSTREAMS)
        def _(j):
            gather_copy(b, j).start()

    def wait_gathers(b):
        @pl.loop(0, GSTREAMS)
        def _(j):
            gather_copy(b, j).wait()

    @pl.when(lax.axis_index("s") == 0)
    def _():
        pltpu.async_copy(table_hbm, table_sh, sem_t).wait()

    bias_copy = pltpu.async_copy(
        bias_hbm.at[pl.ds(base_row, ROWS_PER_WORKER)], out_v, sem_t)
    start_cols(0, 0)
    start_values(0, 0)
    start_cols(1, 1)
    start_values(1, 1)
    bias_copy.wait()
    plsc.subcore_barrier()

    wait_cols(0, 0)
    issue_gathers(0)

    last_mask = lax.iota(jnp.int32, LANES) == (LANES - 1)

    @pl.loop(0, NUM_CHUNKS, step=2)
    def _chunk(ci):
      for b in range(2):
        c = ci + b
        nb = b ^ 1

        @pl.when(c + 1 < NUM_CHUNKS)
        def _():
            wait_cols(c + 1, nb)
            issue_gathers(nb)

        wait_gathers(b)
        wait_values(c, b)

        @pl.when(c + 2 < NUM_CHUNKS)
        def _():
            start_cols(c + 2, b)

        row0 = c * CHUNK_ROWS
        boff = b * CHUNK_NNZ

        @plsc.parallel_loop(0, CHUNK_ROWS, unroll=4)
        def _row(r):
            base = boff + r * NNZ_PER_ROW
            g0 = gath_v[pl.ds(base, LANES)]
            g1 = gath_v[pl.ds(base + LANES, LANES)]
            g2 = gath_v[pl.ds(base + 2 * LANES, LANES)]
            g3 = gath_v[pl.ds(base + 3 * LANES, LANES)]
            v0 = values_v[pl.ds(base, LANES)]
            v1 = values_v[pl.ds(base + LANES, LANES)]
            v2 = values_v[pl.ds(base + 2 * LANES, LANES)]
            v3 = values_v[pl.ds(base + 3 * LANES, LANES)]
            acc = (g0 * v0 + g1 * v1) + (g2 * v2 + g3 * v3)
            cum = plsc.cumsum(acc)
            plsc.addupdate_scatter(
                out_v, [jnp.full((LANES,), row0 + r, jnp.int32)], cum,
                mask=last_mask)

        @pl.when(c + 2 < NUM_CHUNKS)
        def _():
            start_values(c + 2, b)

    pltpu.async_copy(out_v, out_hbm.at[pl.ds(base_row, ROWS_PER_WORKER)],
                     sem_t).wait()


@jax.jit
def _spmv(table, cols, values, bias):
    mesh = plsc.VectorSubcoreMesh(core_axis_name="c", subcore_axis_name="s")
    cp = pltpu.CompilerParams()
    if "needs_layout_passes" in pltpu.CompilerParams.__dataclass_fields__:
        cp = dataclasses.replace(cp, needs_layout_passes=False)
    kern = pl.kernel(
        _spmv_kernel,
        out_type=jax.ShapeDtypeStruct((N_ROWS,), jnp.float32),
        mesh=mesh,
        scratch_types=[
            pltpu.MemorySpace.VMEM_SHARED((N_COLS,), jnp.float32),
            pltpu.VMEM((2 * CHUNK_NNZ,), jnp.int32),
            pltpu.VMEM((2 * CHUNK_NNZ,), jnp.float32),
            pltpu.VMEM((2 * CHUNK_NNZ,), jnp.float32),
            pltpu.VMEM((ROWS_PER_WORKER,), jnp.float32),
            pltpu.SemaphoreType.DMA,
            pltpu.SemaphoreType.DMA,
            pltpu.SemaphoreType.DMA,
            pltpu.SemaphoreType.DMA,
            pltpu.SemaphoreType.DMA,
            pltpu.SemaphoreType.DMA,
            pltpu.SemaphoreType.DMA,
        ],
        compiler_params=cp,
    )
    return kern(table, cols, values, bias)


def kernel(layer_input, rows, cols, values, bias):
    del rows
    table = layer_input.reshape(N_COLS)
    return _spmv(table, cols, values, bias)


# speedup vs baseline: 1.5780x; 1.5780x over previous
import dataclasses

import jax
import jax.numpy as jnp
from jax import lax
from jax.experimental import pallas as pl
from jax.experimental.pallas import tpu as pltpu
from jax.experimental.pallas import tpu_sc as plsc

N_ROWS = 65536
N_COLS = 65536
NNZ_PER_ROW = 64

NUM_WORKERS = 32
ROWS_PER_WORKER = N_ROWS // NUM_WORKERS
CHUNK_ROWS = 128
NUM_CHUNKS = ROWS_PER_WORKER // CHUNK_ROWS
CHUNK_NNZ = CHUNK_ROWS * NNZ_PER_ROW
LANES = 16


def _spmv_kernel(table_hbm, cols_hbm, values_hbm, bias_hbm, out_hbm,
                 table_sh, table_v, cols_v, values_v, out_v, sem_t, sems):
    sid = lax.axis_index("s")
    wid = sid * 2 + lax.axis_index("c")
    base_row = wid * ROWS_PER_WORKER

    def chunk_slices(c):
        nz0 = (base_row + c * CHUNK_ROWS) * NNZ_PER_ROW
        return (cols_hbm.at[pl.ds(nz0, CHUNK_NNZ)],
                values_hbm.at[pl.ds(nz0, CHUNK_NNZ)])

    def start_in(c, b):
        cols_sl, values_sl = chunk_slices(c)
        dst = pl.ds(b * CHUNK_NNZ, CHUNK_NNZ)
        pltpu.async_copy(cols_sl, cols_v.at[dst], sems.at[b])
        pltpu.async_copy(values_sl, values_v.at[dst], sems.at[b])

    def wait_in(c, b):
        cols_sl, values_sl = chunk_slices(c)
        dst = pl.ds(b * CHUNK_NNZ, CHUNK_NNZ)
        pltpu.make_async_copy(cols_sl, cols_v.at[dst], sems.at[b]).wait()
        pltpu.make_async_copy(values_sl, values_v.at[dst], sems.at[b]).wait()

    TSLICE = N_COLS // 16
    toff = sid * TSLICE
    bias_copy = pltpu.async_copy(
        bias_hbm.at[pl.ds(base_row, ROWS_PER_WORKER)], out_v, sem_t)
    pltpu.async_copy(table_hbm.at[pl.ds(toff, TSLICE)],
                     table_sh.at[pl.ds(toff, TSLICE)], sem_t).wait()
    start_in(0, 0)
    plsc.subcore_barrier()
    table_copy = pltpu.async_copy(table_sh, table_v, sem_t)
    bias_copy.wait()
    table_copy.wait()

    last_mask = lax.iota(jnp.int32, LANES) == (LANES - 1)

    @pl.loop(0, NUM_CHUNKS)
    def _chunk(c):
        b = lax.bitwise_and(c, 1)
        nxt = c + 1

        @pl.when(nxt < NUM_CHUNKS)
        def _():
            start_in(nxt, 1 - b)

        wait_in(c, b)
        row0 = c * CHUNK_ROWS
        boff = b * CHUNK_NNZ

        @plsc.parallel_loop(0, CHUNK_ROWS, unroll=4)
        def _row(r):
            base = boff + r * NNZ_PER_ROW
            cbuf = cols_v
            vbuf = values_v
            g0 = plsc.load_gather(table_v, [cbuf[pl.ds(base, LANES)]])
            g1 = plsc.load_gather(table_v,
                                  [cbuf[pl.ds(base + LANES, LANES)]])
            g2 = plsc.load_gather(table_v,
                                  [cbuf[pl.ds(base + 2 * LANES, LANES)]])
            g3 = plsc.load_gather(table_v,
                                  [cbuf[pl.ds(base + 3 * LANES, LANES)]])
            v0 = vbuf[pl.ds(base, LANES)]
            v1 = vbuf[pl.ds(base + LANES, LANES)]
            v2 = vbuf[pl.ds(base + 2 * LANES, LANES)]
            v3 = vbuf[pl.ds(base + 3 * LANES, LANES)]
            acc = (g0 * v0 + g1 * v1) + (g2 * v2 + g3 * v3)
            cum = plsc.cumsum(acc)
            plsc.addupdate_scatter(
                out_v, [jnp.full((LANES,), row0 + r, jnp.int32)], cum,
                mask=last_mask)

    pltpu.async_copy(out_v, out_hbm.at[pl.ds(base_row, ROWS_PER_WORKER)],
                     sem_t).wait()


@jax.jit
def _spmv(table, cols, values, bias):
    mesh = plsc.VectorSubcoreMesh(core_axis_name="c", subcore_axis_name="s")
    cp = pltpu.CompilerParams()
    if "needs_layout_passes" in pltpu.CompilerParams.__dataclass_fields__:
        cp = dataclasses.replace(cp, needs_layout_passes=False)
    kern = pl.kernel(
        _spmv_kernel,
        out_type=jax.ShapeDtypeStruct((N_ROWS,), jnp.float32),
        mesh=mesh,
        scratch_types=[
            pltpu.MemorySpace.VMEM_SHARED((N_COLS,), jnp.float32),
            pltpu.VMEM((N_COLS,), jnp.float32),
            pltpu.VMEM((2 * CHUNK_NNZ,), jnp.int32),
            pltpu.VMEM((2 * CHUNK_NNZ,), jnp.float32),
            pltpu.VMEM((ROWS_PER_WORKER,), jnp.float32),
            pltpu.SemaphoreType.DMA,
            pltpu.SemaphoreType.DMA((2,)),
        ],
        compiler_params=cp,
    )
    return kern(table, cols, values, bias)


def kernel(layer_input, rows, cols, values, bias):
    del rows
    table = layer_input.reshape(N_COLS)
    return _spmv(table, cols, values, bias)
